# Initial kernel scaffold; baseline (speedup 1.0000x reference)
import jax
import jax.numpy as jnp
from jax.experimental import pallas as pl


def kernel(x, edge_index, W_l, b_l, W_r, b_r, att, bias):
    raise NotImplementedError("write your pallas kernel here")


# speedup vs baseline: 5.2872x; 5.2872x over previous
import jax
import jax.numpy as jnp
from jax import lax
from jax.experimental import pallas as pl
from jax.experimental.pallas import tpu as pltpu
from jax.experimental.pallas import tpu_sc as plsc

N = 10000
F = 128
M = 10240
E = 320000
NC, NS, L = 2, 16, 16
NW = NC * NS
K = 64
NCHUNK = 160
EPAD = NW * NCHUNK * K
NB = F // L
ZR = 64
BM = 1024
NEG_SLOPE = 0.2


def _proj_body(x_ref, wl_ref, wr_ref, bl_ref, br_ref, pj_ref):
    xb = x_ref[...]
    pj_ref[0] = jnp.dot(xb, wl_ref[...], preferred_element_type=jnp.float32) + bl_ref[...]
    pj_ref[1] = jnp.dot(xb, wr_ref[...], preferred_element_type=jnp.float32) + br_ref[...]


_proj = pl.pallas_call(
    _proj_body,
    grid=(M // BM,),
    in_specs=[
        pl.BlockSpec((BM, F), lambda i: (i, 0)),
        pl.BlockSpec((F, F), lambda i: (0, 0)),
        pl.BlockSpec((F, F), lambda i: (0, 0)),
        pl.BlockSpec((1, F), lambda i: (0, 0)),
        pl.BlockSpec((1, F), lambda i: (0, 0)),
    ],
    out_specs=pl.BlockSpec((2, BM, F), lambda i: (0, i, 0)),
    out_shape=jax.ShapeDtypeStruct((2, M, F), jnp.float32),
)


def _edge_kernel(xcat_hbm, att_hbm, gidx_hbm, dst_hbm, acc_hbm, den_hbm,
                 idx2, dst_v, gbuf, msrc, p_buf, denom_v, att_v, zbuf,
                 acc_sh, sem1):
    cid = lax.axis_index("c")
    sid = lax.axis_index("s")
    wid = cid * NS + sid

    pltpu.sync_copy(att_hbm, att_v)

    zeros16 = jnp.zeros((L,), jnp.float32)

    def _zden(r, c):
        denom_v[pl.ds(r * L, L)] = zeros16
        return c

    lax.fori_loop(0, M // L, _zden, 0)

    def _zrow(r, c):
        for b in range(NB):
            zbuf[r, pl.ds(b * L, L)] = zeros16
        return c

    lax.fori_loop(0, ZR, _zrow, 0)
    rows_per_tile = M // NS
    r0 = sid * rows_per_tile
    for i in range(rows_per_tile // ZR):
        pltpu.sync_copy(zbuf, acc_sh.at[pl.ds(r0 + i * ZR, ZR)])
    plsc.subcore_barrier()

    att_blk = [att_v[pl.ds(b * L, L)] for b in range(NB)]
    lanes = lax.iota(jnp.int32, L)
    lane_masks = [lanes == ln for ln in range(L)]

    def _chunk(j, c):
        pltpu.sync_copy(gidx_hbm.at[wid, j], idx2)
        pltpu.sync_copy(dst_hbm.at[wid, j], dst_v)

        def _gath(t, cc):
            pltpu.async_copy(xcat_hbm.at[idx2.at[t]], gbuf.at[t], sem1).wait()
            return cc

        lax.fori_loop(0, 2, _gath, 0)

        def _edge(e, cc):
            xl_rows = []
            acc16 = jnp.zeros((L,), jnp.float32)
            for b in range(NB):
                xlb = gbuf[0, e, pl.ds(b * L, L)]
                xrb = gbuf[1, e, pl.ds(b * L, L)]
                xl_rows.append(xlb)
                s = xlb + xrb
                lrelu = jnp.maximum(s, NEG_SLOPE * s)
                acc16 = acc16 + lrelu * att_blk[b]
            tot = jnp.sum(acc16)
            p16 = jnp.exp(lax.broadcast(tot, (L,)))
            for b in range(NB):
                msrc[e, pl.ds(b * L, L)] = xl_rows[b] * p16
            plsc.store_scatter(p_buf, [lax.broadcast(e, (L,))], p16,
                               mask=lane_masks[0])
            return cc

        lax.fori_loop(0, K, _edge, 0)

        for g in range(K // L):
            dst16 = dst_v[pl.ds(g * L, L)]
            p16g = p_buf[pl.ds(g * L, L)]
            for ln in range(L):
                plsc.addupdate_scatter(denom_v, [dst16], p16g,
                                       mask=lane_masks[ln])

        pltpu.sync_copy(msrc, acc_sh.at[dst_v], add=True)
        return c

    lax.fori_loop(0, NCHUNK, _chunk, 0)

    pltpu.sync_copy(denom_v, den_hbm.at[wid])
    plsc.subcore_barrier()
    pltpu.sync_copy(acc_sh.at[pl.ds(r0, rows_per_tile)],
                    acc_hbm.at[cid, pl.ds(r0, rows_per_tile)])


_edges = pl.kernel(
    _edge_kernel,
    out_type=(jax.ShapeDtypeStruct((NC, M, F), jnp.float32),
              jax.ShapeDtypeStruct((NW, M), jnp.float32)),
    mesh=plsc.VectorSubcoreMesh(core_axis_name="c", subcore_axis_name="s"),
    compiler_params=pltpu.CompilerParams(needs_layout_passes=False),
    scratch_types=[
        pltpu.VMEM((2, K), jnp.int32),
        pltpu.VMEM((K,), jnp.int32),
        pltpu.VMEM((2, K, F), jnp.float32),
        pltpu.VMEM((K, F), jnp.float32),
        pltpu.VMEM((K,), jnp.float32),
        pltpu.VMEM((M,), jnp.float32),
        pltpu.VMEM((F,), jnp.float32),
        pltpu.VMEM((ZR, F), jnp.float32),
        pltpu.VMEM_SHARED((M, F), jnp.float32),
        pltpu.SemaphoreType.DMA,
    ],
)


def _final_body(acc_ref, den_ref, pj_ref, att_ref, bias_ref, out_ref):
    xl = pj_ref[0]
    xr = pj_ref[1]
    s = xl + xr
    lrelu = jnp.maximum(s, NEG_SLOPE * s)
    sa = jnp.exp(jnp.sum(lrelu * att_ref[...], axis=1, keepdims=True))
    num = acc_ref[0] + acc_ref[1] + sa * xl
    den = jnp.sum(den_ref[...], axis=0)[:, None] + sa
    out_ref[...] = num / (den + 1e-16) + bias_ref[...]


_final = pl.pallas_call(
    _final_body,
    grid=(M // BM,),
    in_specs=[
        pl.BlockSpec((NC, BM, F), lambda i: (0, i, 0)),
        pl.BlockSpec((NW, BM), lambda i: (0, i)),
        pl.BlockSpec((2, BM, F), lambda i: (0, i, 0)),
        pl.BlockSpec((1, F), lambda i: (0, 0)),
        pl.BlockSpec((1, F), lambda i: (0, 0)),
    ],
    out_specs=pl.BlockSpec((BM, F), lambda i: (i, 0)),
    out_shape=jax.ShapeDtypeStruct((M, F), jnp.float32),
)


def kernel(x, edge_index, W_l, b_l, W_r, b_r, att, bias):
    xp = jnp.concatenate([x, jnp.zeros((M - N, F), jnp.float32)], axis=0)
    src = edge_index[0].astype(jnp.int32)
    dst = edge_index[1].astype(jnp.int32)
    padv = jnp.full((EPAD - E,), N, jnp.int32)
    srcp = jnp.concatenate([src, padv]).reshape(NW, NCHUNK, 1, K)
    dstp = jnp.concatenate([dst, padv]).reshape(NW, NCHUNK, K)
    gidx = jnp.concatenate([srcp, dstp.reshape(NW, NCHUNK, 1, K) + M], axis=2)
    att1 = att.reshape(F)
    att2 = att.reshape(1, F)
    bl2 = b_l.reshape(1, F)
    br2 = b_r.reshape(1, F)
    bias2 = bias.reshape(1, F)

    pj = _proj(xp, W_l, W_r, bl2, br2)
    xcat = pj.reshape(2 * M, F)
    acc, den = _edges(xcat, att1, gidx, dstp)
    out = _final(acc, den, pj, att2, bias2)
    return out[:N]
